# baseline (speedup 1.0000x reference)
import functools

import jax
import jax.numpy as jnp
from jax import lax
from jax.experimental import pallas as pl
from jax.experimental.pallas import tpu as pltpu
from jax.experimental.pallas import tpu_sc as plsc

N = 10000
E = 160000
D = 256
H = 256
C = 64
DELTA_CONST = 2.8332133440562162

BN = 1000

NW = 32
NODES_PER = 80
PASSES = 4
NBUCKET = NW * PASSES
N_PAD = NBUCKET * NODES_PER
CE = 3200
GB = 64
MBUF = CE + 2 * GB
CAP = 2048
CEF = 1280
CE_S = 3200
MBUF_S = CE_S + 2 * GB
MBUF_R = CAP + 2 * GB
LROW = CAP + 2 * GB


def _prep_body(h_ref, w_ref, b_ref, a_ref, bout_ref):
    acc = jnp.dot(h_ref[...], w_ref[...], preferred_element_type=jnp.float32)
    a_ref[...] = acc[:, :D]
    bout_ref[...] = acc[:, D:] + b_ref[...]


def _prep(h, M_W, M_b):
    W2 = jnp.concatenate([M_W[:D], M_W[D:]], axis=1)
    return pl.pallas_call(
        _prep_body,
        grid=(N // BN,),
        in_specs=[
            pl.BlockSpec((BN, D), lambda i: (i, 0)),
            pl.BlockSpec((D, 2 * D), lambda i: (0, 0)),
            pl.BlockSpec((1, D), lambda i: (0, 0)),
        ],
        out_specs=[
            pl.BlockSpec((BN, D), lambda i: (i, 0)),
            pl.BlockSpec((BN, D), lambda i: (i, 0)),
        ],
        out_shape=[
            jax.ShapeDtypeStruct((N, D), jnp.float32),
            jax.ShapeDtypeStruct((N, D), jnp.float32),
        ],
    )(h, W2, M_b.reshape(1, D))


def _stats_common(a_hbm, mloc, msrc, rowbuf, st_s, st_q, st_mx, st_mn, degv,
                  sem_g):
    def accum_edge_rb(rb, lbase, rbase, j):
        local = mloc[pl.ds(lbase + j, 16)][0]
        for c in range(D // 16):
            sl = pl.ds(c * 16, 16)
            r = rb[rbase + j, sl]
            plsc.addupdate(st_s.at[local, sl], r)
            plsc.addupdate(st_q.at[local, sl], r * r)
            st_mx[local, sl] = jnp.maximum(st_mx[local, sl], r)
            st_mn[local, sl] = jnp.minimum(st_mn[local, sl], r)
        one_hot = jnp.where(lax.iota(jnp.int32, 16) == 0, 1.0, 0.0)
        plsc.addupdate(degv.at[pl.ds(local * 16, 16)], one_hot)

    def accum_edge(base, j):
        accum_edge_rb(rowbuf, base, 0, j)

    def gather_batch(base):
        pltpu.async_copy(a_hbm.at[msrc.at[pl.ds(base, GB)]], rowbuf,
                         sem_g).wait()

    def zero_stats():
        def zero_row(rr, _):
            for c in range(D // 16):
                sl = pl.ds(c * 16, 16)
                st_s[rr, sl] = jnp.zeros((16,), jnp.float32)
                st_q[rr, sl] = jnp.zeros((16,), jnp.float32)
                st_mx[rr, sl] = jnp.full((16,), -3.0e38, jnp.float32)
                st_mn[rr, sl] = jnp.full((16,), 3.0e38, jnp.float32)
            degv[pl.ds(rr * 16, 16)] = jnp.zeros((16,), jnp.float32)
            return 0

        lax.fori_loop(0, NODES_PER, zero_row, 0)

    return accum_edge, accum_edge_rb, gather_batch, zero_stats


def _write_stats(lo, s_hbm, q_hbm, mx_hbm, mn_hbm, deg_hbm,
                 st_s, st_q, st_mx, st_mn, degv, sem_w):
    rows = pl.ds(lo, NODES_PER)
    pltpu.async_copy(st_s, s_hbm.at[rows], sem_w)
    pltpu.async_copy(st_q, q_hbm.at[rows], sem_w)
    pltpu.async_copy(st_mx, mx_hbm.at[rows], sem_w)
    pltpu.async_copy(st_mn, mn_hbm.at[rows], sem_w)
    pltpu.async_copy(degv, deg_hbm.at[pl.ds(lo * 16, NODES_PER * 16)], sem_w)
    pltpu.make_async_copy(st_s, s_hbm.at[rows], sem_w).wait()
    pltpu.make_async_copy(st_q, q_hbm.at[rows], sem_w).wait()
    pltpu.make_async_copy(st_mx, mx_hbm.at[rows], sem_w).wait()
    pltpu.make_async_copy(st_mn, mn_hbm.at[rows], sem_w).wait()
    pltpu.make_async_copy(degv, deg_hbm.at[pl.ds(lo * 16, NODES_PER * 16)],
                          sem_w).wait()


def _scan_body(edge_hbm, lsrc_hbm, lloc_hbm, cnt_hbm,
               ebuf0, ebuf1,
               ms0, ms1, ms2, ms3, ml0, ml1, ml2, ml3, cntbuf,
               sem_e0, sem_e1, sem_w):
    wid = lax.axis_index("s") * 2 + lax.axis_index("c")
    NCH = E // CE_S
    msrcs = [ms0, ms1, ms2, ms3]
    mlocs = [ml0, ml1, ml2, ml3]

    def start_load(k, ebuf, sem):
        pltpu.async_copy(edge_hbm.at[:, pl.ds(k * CE_S, CE_S)], ebuf, sem)

    def wait_load(ebuf, sem):
        pltpu.make_async_copy(edge_hbm.at[:, pl.ds(0, CE_S)], ebuf, sem).wait()

    def scan_and_drain(ebuf, carry):
        cs = list(carry[:4])
        ws = list(carry[4:])

        def scan_group(g, cs4):
            sl = pl.ds(g * 16, 16)
            d = ebuf[1, sl]
            sv = ebuf[0, sl]
            b16 = (d * 52429) >> 22
            mine = (b16 & 31) == wid
            local = d - b16 * NODES_PER
            out = []
            for pi in range(PASSES):
                cp = cs4[pi]
                mp = mine & ((b16 >> 5) == pi)
                mi = jnp.where(mp, 1, 0)
                incl = plsc.cumsum(mi)
                pos = cp + incl - mi
                plsc.store_scatter(msrcs[pi], [pos], sv, mask=mp)
                plsc.store_scatter(mlocs[pi], [pos], local, mask=mp)
                out.append(cp + incl[15])
            return tuple(out)

        cs = list(lax.fori_loop(0, CE_S // 16, scan_group, tuple(cs)))

        for pi in range(PASSES):
            bucket = pi * NW + wid
            cp, wp = cs[pi], ws[pi]
            nb = cp // GB

            def wr(b, _):
                @pl.when((wp + b) * GB <= CAP)
                def _():
                    pltpu.async_copy(
                        msrcs[pi].at[pl.ds(b * GB, GB)],
                        lsrc_hbm.at[bucket, pl.ds((wp + b) * GB, GB)], sem_w)
                    pltpu.async_copy(
                        mlocs[pi].at[pl.ds(b * GB, GB)],
                        lloc_hbm.at[bucket, pl.ds((wp + b) * GB, GB)], sem_w)
                return 0

            lax.fori_loop(0, nb, wr, 0)

            def wt(b, _):
                @pl.when((wp + b) * GB <= CAP)
                def _():
                    pltpu.make_async_copy(
                        msrcs[pi].at[pl.ds(0, GB)],
                        lsrc_hbm.at[bucket, pl.ds(0, GB)], sem_w).wait()
                    pltpu.make_async_copy(
                        mlocs[pi].at[pl.ds(0, GB)],
                        lloc_hbm.at[bucket, pl.ds(0, GB)], sem_w).wait()
                return 0

            lax.fori_loop(0, nb, wt, 0)
            for t in range(GB // 16):
                sl_to = pl.ds(t * 16, 16)
                sl_from = pl.ds(nb * GB + t * 16, 16)
                msrcs[pi][sl_to] = msrcs[pi][sl_from]
                mlocs[pi][sl_to] = mlocs[pi][sl_from]
            cs[pi] = cp - nb * GB
            ws[pi] = wp + nb
        return tuple(cs) + tuple(ws)

    start_load(0, ebuf0, sem_e0)

    def pair_step(kk, carry):
        k0 = 2 * kk
        start_load(k0 + 1, ebuf1, sem_e1)
        wait_load(ebuf0, sem_e0)
        carry = scan_and_drain(ebuf0, carry)

        @pl.when(k0 + 2 < NCH)
        def _():
            start_load(k0 + 2, ebuf0, sem_e0)

        wait_load(ebuf1, sem_e1)
        return scan_and_drain(ebuf1, carry)

    zero = jnp.int32(0)
    carry = lax.fori_loop(0, NCH // 2, pair_step, (zero,) * 8)

    for pi in range(PASSES):
        bucket = pi * NW + wid
        cp = carry[pi]
        wp = carry[4 + pi]
        for t in range(GB // 16):
            msrcs[pi][pl.ds(cp + t * 16, 16)] = jnp.zeros((16,), jnp.int32)

        @pl.when(cp > 0)
        def _():
            @pl.when(wp * GB <= CAP)
            def _():
                pltpu.async_copy(
                    msrcs[pi].at[pl.ds(0, GB)],
                    lsrc_hbm.at[bucket, pl.ds(wp * GB, GB)], sem_w)
                pltpu.async_copy(
                    mlocs[pi].at[pl.ds(0, GB)],
                    lloc_hbm.at[bucket, pl.ds(wp * GB, GB)], sem_w)
                pltpu.make_async_copy(
                    msrcs[pi].at[pl.ds(0, GB)],
                    lsrc_hbm.at[bucket, pl.ds(0, GB)], sem_w).wait()
                pltpu.make_async_copy(
                    mlocs[pi].at[pl.ds(0, GB)],
                    lloc_hbm.at[bucket, pl.ds(0, GB)], sem_w).wait()

        count = wp * GB + cp
        cntbuf[pl.ds(0, 16)] = jnp.broadcast_to(count, (16,))
        pltpu.sync_copy(cntbuf, cnt_hbm.at[pl.ds(bucket * 16, 16)])


_edge_scan = pl.kernel(
    _scan_body,
    out_type=[
        jax.ShapeDtypeStruct((NBUCKET, LROW), jnp.int32),
        jax.ShapeDtypeStruct((NBUCKET, LROW), jnp.int32),
        jax.ShapeDtypeStruct((NBUCKET * 16,), jnp.int32),
    ],
    compiler_params=pltpu.CompilerParams(needs_layout_passes=False),
    mesh=plsc.VectorSubcoreMesh(core_axis_name="c", subcore_axis_name="s"),
    scratch_types=[
        pltpu.VMEM((2, CE_S), jnp.int32),
        pltpu.VMEM((2, CE_S), jnp.int32),
        pltpu.VMEM((MBUF_S,), jnp.int32),
        pltpu.VMEM((MBUF_S,), jnp.int32),
        pltpu.VMEM((MBUF_S,), jnp.int32),
        pltpu.VMEM((MBUF_S,), jnp.int32),
        pltpu.VMEM((MBUF_S,), jnp.int32),
        pltpu.VMEM((MBUF_S,), jnp.int32),
        pltpu.VMEM((MBUF_S,), jnp.int32),
        pltpu.VMEM((MBUF_S,), jnp.int32),
        pltpu.VMEM((16,), jnp.int32),
        pltpu.SemaphoreType.DMA,
        pltpu.SemaphoreType.DMA,
        pltpu.SemaphoreType.DMA,
    ],
)


def _replay_body(a_hbm, edge_hbm, lsrc_hbm, lloc_hbm, cnt_hbm,
                 s_hbm, q_hbm, mx_hbm, mn_hbm, deg_hbm,
                 ebuf0, mloc, msrc, rowbuf, rowbuf1,
                 st_s, st_q, st_mx, st_mn, degv, cntbuf,
                 sem_e0, sem_g, sem_g1, sem_l):
    wid = lax.axis_index("s") * 2 + lax.axis_index("c")
    NCH = E // CE
    accum_edge, accum_edge_rb, gather_batch, zero_stats = _stats_common(
        a_hbm, mloc, msrc, rowbuf, st_s, st_q, st_mx, st_mn, degv, sem_g)

    def gather_to(rb, sem, base):
        pltpu.async_copy(a_hbm.at[msrc.at[pl.ds(base, GB)]], rb, sem)

    def wait_gather(rb, sem):
        pltpu.make_async_copy(a_hbm.at[msrc.at[pl.ds(0, GB)]], rb, sem).wait()

    def accum_block(rb, lbase):
        def step(j, _):
            accum_edge_rb(rb, lbase, 0, j)
            return 0

        lax.fori_loop(0, GB, step, 0, unroll=2)

    def run_pass(p, _):
        bucket = p * NW + wid
        lo = bucket * NODES_PER
        zero_stats()
        pltpu.sync_copy(cnt_hbm.at[pl.ds(bucket * 16, 16)], cntbuf)
        count = cntbuf[pl.ds(0, 16)][0]

        @pl.when(count <= CAP)
        def _():
            def issue_blk(b, _):
                sl = pl.ds(b * 128, 128)
                pltpu.async_copy(lsrc_hbm.at[bucket, sl], msrc.at[sl], sem_l)
                pltpu.async_copy(lloc_hbm.at[bucket, sl], mloc.at[sl], sem_l)
                return 0

            lax.fori_loop(0, LROW // 128, issue_blk, 0)

            def wait_blk(b, _):
                sl0 = pl.ds(0, 128)
                pltpu.make_async_copy(lsrc_hbm.at[bucket, sl0],
                                      msrc.at[sl0], sem_l).wait()
                pltpu.make_async_copy(lloc_hbm.at[bucket, sl0],
                                      mloc.at[sl0], sem_l).wait()
                return 0

            lax.fori_loop(0, LROW // 128, wait_blk, 0)
            nfull = count // GB
            rem = count - nfull * GB

            @pl.when(nfull > 0)
            def _():
                gather_to(rowbuf, sem_g, 0)

            def pairs(bb, _):
                b0 = 2 * bb
                b1 = b0 + 1

                @pl.when(b1 < nfull)
                def _():
                    gather_to(rowbuf1, sem_g1, b1 * GB)

                wait_gather(rowbuf, sem_g)
                accum_block(rowbuf, b0 * GB)

                @pl.when(b1 + 1 < nfull)
                def _():
                    gather_to(rowbuf, sem_g, (b1 + 1) * GB)

                @pl.when(b1 < nfull)
                def _():
                    wait_gather(rowbuf1, sem_g1)
                    accum_block(rowbuf1, b1 * GB)

                return 0

            lax.fori_loop(0, (nfull + 1) // 2, pairs, 0)

            @pl.when(rem > 0)
            def _():
                gather_to(rowbuf, sem_g, nfull * GB)
                wait_gather(rowbuf, sem_g)

                def step(j, _):
                    accum_edge_rb(rowbuf, nfull * GB, 0, j)
                    return 0

                lax.fori_loop(0, rem, step, 0, unroll=False)

        @pl.when(count > CAP)
        def _():
            def scan_group_mk(g, cursor):
                sl = pl.ds(g * 16, 16)
                d = ebuf0[1, sl]
                sv = ebuf0[0, sl]
                m = (d >= lo) & (d < lo + NODES_PER)
                mi = jnp.where(m, 1, 0)
                incl = plsc.cumsum(mi)
                pos = cursor + incl - mi
                plsc.store_scatter(msrc, [pos], sv, mask=m)
                plsc.store_scatter(mloc, [pos], d - lo, mask=m)
                return cursor + incl[15]

            def drain_full(b, _):
                base = b * GB
                gather_batch(base)

                def step(j, _):
                    accum_edge(base, j)
                    return 0

                lax.fori_loop(0, GB, step, 0, unroll=2)
                return 0

            def chunk_step(k, cursor):
                pltpu.sync_copy(edge_hbm.at[:, pl.ds(k * CEF, CEF)], ebuf0)
                cursor = lax.fori_loop(0, CEF // 16, scan_group_mk, cursor)
                nb = cursor // GB
                lax.fori_loop(0, nb, drain_full, 0)
                for t in range(GB // 16):
                    sl_to = pl.ds(t * 16, 16)
                    sl_from = pl.ds(nb * GB + t * 16, 16)
                    msrc[sl_to] = msrc[sl_from]
                    mloc[sl_to] = mloc[sl_from]
                return cursor - nb * GB

            cursor = lax.fori_loop(0, E // CEF, chunk_step, jnp.int32(0))
            for t in range(GB // 16):
                msrc[pl.ds(cursor + t * 16, 16)] = jnp.zeros((16,), jnp.int32)
            gather_batch(0)

            def final_step(j, _):
                accum_edge(0, j)
                return 0

            lax.fori_loop(0, cursor, final_step, 0, unroll=False)

        _write_stats(lo, s_hbm, q_hbm, mx_hbm, mn_hbm, deg_hbm,
                     st_s, st_q, st_mx, st_mn, degv, sem_l)
        return 0

    lax.fori_loop(0, PASSES, run_pass, 0)


_STATS_OUT = [
    jax.ShapeDtypeStruct((N_PAD, D), jnp.float32),
    jax.ShapeDtypeStruct((N_PAD, D), jnp.float32),
    jax.ShapeDtypeStruct((N_PAD, D), jnp.float32),
    jax.ShapeDtypeStruct((N_PAD, D), jnp.float32),
    jax.ShapeDtypeStruct((N_PAD * 16,), jnp.float32),
]

_STATS_SCRATCH = [
    pltpu.VMEM((MBUF,), jnp.int32),
    pltpu.VMEM((MBUF,), jnp.int32),
    pltpu.VMEM((GB, D), jnp.float32),
    pltpu.VMEM((NODES_PER, D), jnp.float32),
    pltpu.VMEM((NODES_PER, D), jnp.float32),
    pltpu.VMEM((NODES_PER, D), jnp.float32),
    pltpu.VMEM((NODES_PER, D), jnp.float32),
    pltpu.VMEM((NODES_PER * 16,), jnp.float32),
    pltpu.VMEM((16,), jnp.int32),
]

_stats_replay = pl.kernel(
    _replay_body,
    out_type=_STATS_OUT,
    compiler_params=pltpu.CompilerParams(needs_layout_passes=False),
    mesh=plsc.VectorSubcoreMesh(core_axis_name="c", subcore_axis_name="s"),
    scratch_types=[
        pltpu.VMEM((2, CEF), jnp.int32),
        pltpu.VMEM((MBUF_R,), jnp.int32),
        pltpu.VMEM((MBUF_R,), jnp.int32),
        pltpu.VMEM((GB, D), jnp.float32),
        pltpu.VMEM((GB, D), jnp.float32),
        pltpu.VMEM((NODES_PER, D), jnp.float32),
        pltpu.VMEM((NODES_PER, D), jnp.float32),
        pltpu.VMEM((NODES_PER, D), jnp.float32),
        pltpu.VMEM((NODES_PER, D), jnp.float32),
        pltpu.VMEM((NODES_PER * 16,), jnp.float32),
        pltpu.VMEM((16,), jnp.int32),
        pltpu.SemaphoreType.DMA,
        pltpu.SemaphoreType.DMA,
        pltpu.SemaphoreType.DMA,
        pltpu.SemaphoreType.DMA,
    ],
)


def _stats_from_lists(A, edge_index, lists):
    lsrc, lloc, cnt = lists
    S, Q, Mx, Mn, deg = _stats_replay(A, edge_index, lsrc, lloc, cnt)
    return S[:N], Q[:N], Mx[:N], Mn[:N], deg.reshape(N_PAD, 16)[:N, 0]


def _stats_blocks(h_ref, b_ref, s_ref, q_ref, mx_ref, mn_ref, deg_ref):
    deg = deg_ref[...]
    B = b_ref[...]
    S = s_ref[...]
    denom = jnp.maximum(deg, 1.0)
    mean = (S + deg * B) / denom
    sq = (q_ref[...] + 2.0 * B * S + deg * B * B) / denom
    std = jnp.sqrt(jnp.maximum(sq - mean * mean, 0.0) + 1e-5)
    pos = deg > 0.0
    mx = jnp.where(pos, mx_ref[...] + B, 0.0)
    mn = jnp.where(pos, mn_ref[...] + B, 0.0)
    logd = jnp.log(deg + 1.0)
    amp = logd / DELTA_CONST
    att = jnp.where(pos, DELTA_CONST / jnp.maximum(logd, 1e-12), 1.0)
    h = h_ref[...]
    return h, mean, mx, mn, std, amp, att


def _pna_update(h, mean, mx, mn, std, amp, att, U, Ub, mixW, mixb):
    def dot(x, w):
        return jnp.dot(x, w, preferred_element_type=jnp.float32)

    acc = dot(h, U[0:D])
    accI = dot(mean, U[D:2 * D]) + dot(mx, U[2 * D:3 * D]) \
        + dot(mn, U[3 * D:4 * D]) + dot(std, U[4 * D:5 * D])
    accA = dot(mean, U[5 * D:6 * D]) + dot(mx, U[6 * D:7 * D]) \
        + dot(mn, U[7 * D:8 * D]) + dot(std, U[8 * D:9 * D])
    accT = dot(mean, U[9 * D:10 * D]) + dot(mx, U[10 * D:11 * D]) \
        + dot(mn, U[11 * D:12 * D]) + dot(std, U[12 * D:13 * D])
    pre = acc + accI + amp * accA + att * accT + Ub
    out = dot(pre, mixW) + mixb
    out = jnp.where(out >= 0.0, out, 0.01 * out) + h
    return out


def _combine1_body(h_ref, b_ref, s_ref, q_ref, mx_ref, mn_ref, deg_ref,
                   u_ref, ub_ref, mixw_ref, mixb_ref, out_ref):
    h, mean, mx, mn, std, amp, att = _stats_blocks(
        h_ref, b_ref, s_ref, q_ref, mx_ref, mn_ref, deg_ref)
    out = _pna_update(h, mean, mx, mn, std, amp, att,
                      u_ref[...], ub_ref[...], mixw_ref[...], mixb_ref[...])
    out_ref[...] = jnp.maximum(out, 0.0)


def _combine2_body(h_ref, b_ref, s_ref, q_ref, mx_ref, mn_ref, deg_ref,
                   u_ref, ub_ref, mixw_ref, mixb_ref, fcw_ref, fcb_ref,
                   out_ref, acc_ref):
    i = pl.program_id(0)
    h, mean, mx, mn, std, amp, att = _stats_blocks(
        h_ref, b_ref, s_ref, q_ref, mx_ref, mn_ref, deg_ref)
    out = _pna_update(h, mean, mx, mn, std, amp, att,
                      u_ref[...], ub_ref[...], mixw_ref[...], mixb_ref[...])
    partial = jnp.sum(out, axis=0, keepdims=True)

    @pl.when(i == 0)
    def _():
        acc_ref[...] = partial

    @pl.when(i > 0)
    def _():
        acc_ref[...] = acc_ref[...] + partial

    @pl.when(i == (N // BN) - 1)
    def _():
        hg = acc_ref[...] * (1.0 / N)
        out_ref[...] = jnp.dot(hg, fcw_ref[...],
                               preferred_element_type=jnp.float32) + fcb_ref[...]


def _node_spec():
    return pl.BlockSpec((BN, D), lambda i: (i, 0))


def _fixed(shape):
    return pl.BlockSpec(shape, lambda i: tuple(0 for _ in shape))


def _combine1(h, B, S, Q, Mx, Mn, deg, U_W, U_b, mix_W, mix_b):
    return pl.pallas_call(
        _combine1_body,
        grid=(N // BN,),
        in_specs=[
            _node_spec(), _node_spec(), _node_spec(), _node_spec(),
            _node_spec(), _node_spec(),
            pl.BlockSpec((BN, 1), lambda i: (i, 0)),
            _fixed((13 * D, H)), _fixed((1, H)), _fixed((H, H)), _fixed((1, H)),
        ],
        out_specs=_node_spec(),
        out_shape=jax.ShapeDtypeStruct((N, H), jnp.float32),
    )(h, B, S, Q, Mx, Mn, deg.reshape(N, 1), U_W, U_b.reshape(1, H),
      mix_W, mix_b.reshape(1, H))


def _combine2(h, B, S, Q, Mx, Mn, deg, U_W, U_b, mix_W, mix_b, fc_W, fc_b):
    return pl.pallas_call(
        _combine2_body,
        grid=(N // BN,),
        in_specs=[
            _node_spec(), _node_spec(), _node_spec(), _node_spec(),
            _node_spec(), _node_spec(),
            pl.BlockSpec((BN, 1), lambda i: (i, 0)),
            _fixed((13 * H, H)), _fixed((1, H)), _fixed((H, H)), _fixed((1, H)),
            _fixed((H, C)), _fixed((1, C)),
        ],
        out_specs=_fixed((1, C)),
        out_shape=jax.ShapeDtypeStruct((1, C), jnp.float32),
        scratch_shapes=[pltpu.VMEM((1, H), jnp.float32)],
    )(h, B, S, Q, Mx, Mn, deg.reshape(N, 1), U_W, U_b.reshape(1, H),
      mix_W, mix_b.reshape(1, H), fc_W, fc_b.reshape(1, C))


def kernel(x, edge_index, M1_W, M1_b, U1_W, U1_b, mix1_W, mix1_b,
           M2_W, M2_b, U2_W, U2_b, mix2_W, mix2_b, fc_W, fc_b):
    src = edge_index[0]
    dst = edge_index[1]

    lists = _edge_scan(edge_index)
    A1, B1 = _prep(x, M1_W, M1_b)
    S, Q, Mx, Mn, deg = _stats_from_lists(A1, edge_index, lists)
    h1 = _combine1(x, B1, S, Q, Mx, Mn, deg, U1_W, U1_b, mix1_W, mix1_b)

    A2, B2 = _prep(h1, M2_W, M2_b)
    S, Q, Mx, Mn, deg = _stats_from_lists(A2, edge_index, lists)
    return _combine2(h1, B2, S, Q, Mx, Mn, deg, U2_W, U2_b, mix2_W, mix2_b,
                     fc_W, fc_b)


# speedup vs baseline: 1.0059x; 1.0059x over previous
import functools

import jax
import jax.numpy as jnp
from jax import lax
from jax.experimental import pallas as pl
from jax.experimental.pallas import tpu as pltpu
from jax.experimental.pallas import tpu_sc as plsc

N = 10000
E = 160000
D = 256
H = 256
C = 64
DELTA_CONST = 2.8332133440562162

BN = 1000

NW = 32
NODES_PER = 80
PASSES = 4
NBUCKET = NW * PASSES
N_PAD = NBUCKET * NODES_PER
CE = 3200
GB = 64
MBUF = CE + 2 * GB
CAP = 2048
CEF = 1280
CE_S = 3200
MBUF_S = CE_S + 2 * GB
MBUF_R = CAP + 2 * GB
LROW = CAP + 2 * GB


def _prep_body(h_ref, w_ref, b_ref, a_ref, bout_ref):
    acc = jnp.dot(h_ref[...], w_ref[...], preferred_element_type=jnp.float32)
    a_ref[...] = acc[:, :D]
    bout_ref[...] = acc[:, D:] + b_ref[...]


def _prep(h, M_W, M_b):
    W2 = jnp.concatenate([M_W[:D], M_W[D:]], axis=1)
    return pl.pallas_call(
        _prep_body,
        grid=(N // BN,),
        in_specs=[
            pl.BlockSpec((BN, D), lambda i: (i, 0)),
            pl.BlockSpec((D, 2 * D), lambda i: (0, 0)),
            pl.BlockSpec((1, D), lambda i: (0, 0)),
        ],
        out_specs=[
            pl.BlockSpec((BN, D), lambda i: (i, 0)),
            pl.BlockSpec((BN, D), lambda i: (i, 0)),
        ],
        out_shape=[
            jax.ShapeDtypeStruct((N, D), jnp.float32),
            jax.ShapeDtypeStruct((N, D), jnp.float32),
        ],
    )(h, W2, M_b.reshape(1, D))


def _stats_common(a_hbm, mloc, msrc, rowbuf, st_s, st_q, st_mx, st_mn, degv,
                  sem_g, with_deg=True):
    one_hot = jnp.where(lax.iota(jnp.int32, 16) == 0, 1.0, 0.0)

    def accum_edge_rb(rb, lbase, rbase, j):
        local = mloc[pl.ds(lbase + j, 16)][0]
        for c in range(D // 16):
            sl = pl.ds(c * 16, 16)
            r = rb[rbase + j, sl]
            plsc.addupdate(st_s.at[local, sl], r)
            plsc.addupdate(st_q.at[local, sl], r * r)
            st_mx[local, sl] = jnp.maximum(st_mx[local, sl], r)
            st_mn[local, sl] = jnp.minimum(st_mn[local, sl], r)
        if with_deg:
            plsc.addupdate(degv.at[pl.ds(local * 16, 16)], one_hot)

    def accum_edge(base, j):
        accum_edge_rb(rowbuf, base, 0, j)

    def gather_batch(base):
        pltpu.async_copy(a_hbm.at[msrc.at[pl.ds(base, GB)]], rowbuf,
                         sem_g).wait()

    def zero_stats():
        def zero_row(rr, _):
            for c in range(D // 16):
                sl = pl.ds(c * 16, 16)
                st_s[rr, sl] = jnp.zeros((16,), jnp.float32)
                st_q[rr, sl] = jnp.zeros((16,), jnp.float32)
                st_mx[rr, sl] = jnp.full((16,), -3.0e38, jnp.float32)
                st_mn[rr, sl] = jnp.full((16,), 3.0e38, jnp.float32)
            degv[pl.ds(rr * 16, 16)] = jnp.zeros((16,), jnp.float32)
            return 0

        lax.fori_loop(0, NODES_PER, zero_row, 0)

    return accum_edge, accum_edge_rb, gather_batch, zero_stats


def _write_stats(lo, s_hbm, q_hbm, mx_hbm, mn_hbm, deg_hbm,
                 st_s, st_q, st_mx, st_mn, degv, sem_w, with_deg=True):
    rows = pl.ds(lo, NODES_PER)
    pltpu.async_copy(st_s, s_hbm.at[rows], sem_w)
    pltpu.async_copy(st_q, q_hbm.at[rows], sem_w)
    pltpu.async_copy(st_mx, mx_hbm.at[rows], sem_w)
    pltpu.async_copy(st_mn, mn_hbm.at[rows], sem_w)
    if with_deg:
        pltpu.async_copy(degv, deg_hbm.at[pl.ds(lo * 16, NODES_PER * 16)],
                         sem_w)
    pltpu.make_async_copy(st_s, s_hbm.at[rows], sem_w).wait()
    pltpu.make_async_copy(st_q, q_hbm.at[rows], sem_w).wait()
    pltpu.make_async_copy(st_mx, mx_hbm.at[rows], sem_w).wait()
    pltpu.make_async_copy(st_mn, mn_hbm.at[rows], sem_w).wait()
    if with_deg:
        pltpu.make_async_copy(degv,
                              deg_hbm.at[pl.ds(lo * 16, NODES_PER * 16)],
                              sem_w).wait()


def _scan_body(edge_hbm, lsrc_hbm, lloc_hbm, cnt_hbm,
               ebuf0, ebuf1,
               ms0, ms1, ms2, ms3, ml0, ml1, ml2, ml3, cntbuf,
               sem_e0, sem_e1, sem_w):
    wid = lax.axis_index("s") * 2 + lax.axis_index("c")
    NCH = E // CE_S
    msrcs = [ms0, ms1, ms2, ms3]
    mlocs = [ml0, ml1, ml2, ml3]

    def start_load(k, ebuf, sem):
        pltpu.async_copy(edge_hbm.at[:, pl.ds(k * CE_S, CE_S)], ebuf, sem)

    def wait_load(ebuf, sem):
        pltpu.make_async_copy(edge_hbm.at[:, pl.ds(0, CE_S)], ebuf, sem).wait()

    def scan_and_drain(ebuf, carry):
        cs = list(carry[:4])
        ws = list(carry[4:])

        def scan_group(g, cs4):
            sl = pl.ds(g * 16, 16)
            d = ebuf[1, sl]
            sv = ebuf[0, sl]
            b16 = (d * 52429) >> 22
            mine = (b16 & 31) == wid
            local = d - b16 * NODES_PER
            out = []
            for pi in range(PASSES):
                cp = cs4[pi]
                mp = mine & ((b16 >> 5) == pi)
                mi = jnp.where(mp, 1, 0)
                incl = plsc.cumsum(mi)
                pos = cp + incl - mi
                plsc.store_scatter(msrcs[pi], [pos], sv, mask=mp)
                plsc.store_scatter(mlocs[pi], [pos], local, mask=mp)
                out.append(cp + incl[15])
            return tuple(out)

        cs = list(lax.fori_loop(0, CE_S // 16, scan_group, tuple(cs)))

        for pi in range(PASSES):
            bucket = pi * NW + wid
            cp, wp = cs[pi], ws[pi]
            nb = cp // GB

            def wr(b, _):
                @pl.when((wp + b) * GB <= CAP)
                def _():
                    pltpu.async_copy(
                        msrcs[pi].at[pl.ds(b * GB, GB)],
                        lsrc_hbm.at[bucket, pl.ds((wp + b) * GB, GB)], sem_w)
                    pltpu.async_copy(
                        mlocs[pi].at[pl.ds(b * GB, GB)],
                        lloc_hbm.at[bucket, pl.ds((wp + b) * GB, GB)], sem_w)
                return 0

            lax.fori_loop(0, nb, wr, 0)

            def wt(b, _):
                @pl.when((wp + b) * GB <= CAP)
                def _():
                    pltpu.make_async_copy(
                        msrcs[pi].at[pl.ds(0, GB)],
                        lsrc_hbm.at[bucket, pl.ds(0, GB)], sem_w).wait()
                    pltpu.make_async_copy(
                        mlocs[pi].at[pl.ds(0, GB)],
                        lloc_hbm.at[bucket, pl.ds(0, GB)], sem_w).wait()
                return 0

            lax.fori_loop(0, nb, wt, 0)
            for t in range(GB // 16):
                sl_to = pl.ds(t * 16, 16)
                sl_from = pl.ds(nb * GB + t * 16, 16)
                msrcs[pi][sl_to] = msrcs[pi][sl_from]
                mlocs[pi][sl_to] = mlocs[pi][sl_from]
            cs[pi] = cp - nb * GB
            ws[pi] = wp + nb
        return tuple(cs) + tuple(ws)

    start_load(0, ebuf0, sem_e0)

    def pair_step(kk, carry):
        k0 = 2 * kk
        start_load(k0 + 1, ebuf1, sem_e1)
        wait_load(ebuf0, sem_e0)
        carry = scan_and_drain(ebuf0, carry)

        @pl.when(k0 + 2 < NCH)
        def _():
            start_load(k0 + 2, ebuf0, sem_e0)

        wait_load(ebuf1, sem_e1)
        return scan_and_drain(ebuf1, carry)

    zero = jnp.int32(0)
    carry = lax.fori_loop(0, NCH // 2, pair_step, (zero,) * 8)

    for pi in range(PASSES):
        bucket = pi * NW + wid
        cp = carry[pi]
        wp = carry[4 + pi]
        for t in range(GB // 16):
            msrcs[pi][pl.ds(cp + t * 16, 16)] = jnp.zeros((16,), jnp.int32)

        @pl.when(cp > 0)
        def _():
            @pl.when(wp * GB <= CAP)
            def _():
                pltpu.async_copy(
                    msrcs[pi].at[pl.ds(0, GB)],
                    lsrc_hbm.at[bucket, pl.ds(wp * GB, GB)], sem_w)
                pltpu.async_copy(
                    mlocs[pi].at[pl.ds(0, GB)],
                    lloc_hbm.at[bucket, pl.ds(wp * GB, GB)], sem_w)
                pltpu.make_async_copy(
                    msrcs[pi].at[pl.ds(0, GB)],
                    lsrc_hbm.at[bucket, pl.ds(0, GB)], sem_w).wait()
                pltpu.make_async_copy(
                    mlocs[pi].at[pl.ds(0, GB)],
                    lloc_hbm.at[bucket, pl.ds(0, GB)], sem_w).wait()

        count = wp * GB + cp
        cntbuf[pl.ds(0, 16)] = jnp.broadcast_to(count, (16,))
        pltpu.sync_copy(cntbuf, cnt_hbm.at[pl.ds(bucket * 16, 16)])


_edge_scan = pl.kernel(
    _scan_body,
    out_type=[
        jax.ShapeDtypeStruct((NBUCKET, LROW), jnp.int32),
        jax.ShapeDtypeStruct((NBUCKET, LROW), jnp.int32),
        jax.ShapeDtypeStruct((NBUCKET * 16,), jnp.int32),
    ],
    compiler_params=pltpu.CompilerParams(needs_layout_passes=False),
    mesh=plsc.VectorSubcoreMesh(core_axis_name="c", subcore_axis_name="s"),
    scratch_types=[
        pltpu.VMEM((2, CE_S), jnp.int32),
        pltpu.VMEM((2, CE_S), jnp.int32),
        pltpu.VMEM((MBUF_S,), jnp.int32),
        pltpu.VMEM((MBUF_S,), jnp.int32),
        pltpu.VMEM((MBUF_S,), jnp.int32),
        pltpu.VMEM((MBUF_S,), jnp.int32),
        pltpu.VMEM((MBUF_S,), jnp.int32),
        pltpu.VMEM((MBUF_S,), jnp.int32),
        pltpu.VMEM((MBUF_S,), jnp.int32),
        pltpu.VMEM((MBUF_S,), jnp.int32),
        pltpu.VMEM((16,), jnp.int32),
        pltpu.SemaphoreType.DMA,
        pltpu.SemaphoreType.DMA,
        pltpu.SemaphoreType.DMA,
    ],
)


def _replay_body(a_hbm, edge_hbm, lsrc_hbm, lloc_hbm, cnt_hbm,
                 s_hbm, q_hbm, mx_hbm, mn_hbm, deg_hbm,
                 ebuf0, mloc, msrc, rowbuf, rowbuf1,
                 st_s, st_q, st_mx, st_mn, degv, cntbuf,
                 sem_e0, sem_g, sem_g1, sem_l, with_deg=True):
    wid = lax.axis_index("s") * 2 + lax.axis_index("c")
    NCH = E // CE
    accum_edge, accum_edge_rb, gather_batch, zero_stats = _stats_common(
        a_hbm, mloc, msrc, rowbuf, st_s, st_q, st_mx, st_mn, degv, sem_g,
        with_deg=with_deg)

    def gather_to(rb, sem, base):
        pltpu.async_copy(a_hbm.at[msrc.at[pl.ds(base, GB)]], rb, sem)

    def wait_gather(rb, sem):
        pltpu.make_async_copy(a_hbm.at[msrc.at[pl.ds(0, GB)]], rb, sem).wait()

    def accum_block(rb, lbase):
        def step(j, _):
            accum_edge_rb(rb, lbase, 0, j)
            return 0

        lax.fori_loop(0, GB, step, 0, unroll=2)

    def run_pass(p, _):
        bucket = p * NW + wid
        lo = bucket * NODES_PER
        zero_stats()
        pltpu.sync_copy(cnt_hbm.at[pl.ds(bucket * 16, 16)], cntbuf)
        count = cntbuf[pl.ds(0, 16)][0]

        @pl.when(count <= CAP)
        def _():
            def issue_blk(b, _):
                sl = pl.ds(b * 128, 128)
                pltpu.async_copy(lsrc_hbm.at[bucket, sl], msrc.at[sl], sem_l)
                pltpu.async_copy(lloc_hbm.at[bucket, sl], mloc.at[sl], sem_l)
                return 0

            lax.fori_loop(0, LROW // 128, issue_blk, 0)

            def wait_blk(b, _):
                sl0 = pl.ds(0, 128)
                pltpu.make_async_copy(lsrc_hbm.at[bucket, sl0],
                                      msrc.at[sl0], sem_l).wait()
                pltpu.make_async_copy(lloc_hbm.at[bucket, sl0],
                                      mloc.at[sl0], sem_l).wait()
                return 0

            lax.fori_loop(0, LROW // 128, wait_blk, 0)
            nfull = count // GB
            rem = count - nfull * GB

            @pl.when(nfull > 0)
            def _():
                gather_to(rowbuf, sem_g, 0)

            def pairs(bb, _):
                b0 = 2 * bb
                b1 = b0 + 1

                @pl.when(b1 < nfull)
                def _():
                    gather_to(rowbuf1, sem_g1, b1 * GB)

                wait_gather(rowbuf, sem_g)
                accum_block(rowbuf, b0 * GB)

                @pl.when(b1 + 1 < nfull)
                def _():
                    gather_to(rowbuf, sem_g, (b1 + 1) * GB)

                @pl.when(b1 < nfull)
                def _():
                    wait_gather(rowbuf1, sem_g1)
                    accum_block(rowbuf1, b1 * GB)

                return 0

            lax.fori_loop(0, (nfull + 1) // 2, pairs, 0)

            @pl.when(rem > 0)
            def _():
                gather_to(rowbuf, sem_g, nfull * GB)
                wait_gather(rowbuf, sem_g)

                def step(j, _):
                    accum_edge_rb(rowbuf, nfull * GB, 0, j)
                    return 0

                lax.fori_loop(0, rem, step, 0, unroll=False)

        @pl.when(count > CAP)
        def _():
            def scan_group_mk(g, cursor):
                sl = pl.ds(g * 16, 16)
                d = ebuf0[1, sl]
                sv = ebuf0[0, sl]
                m = (d >= lo) & (d < lo + NODES_PER)
                mi = jnp.where(m, 1, 0)
                incl = plsc.cumsum(mi)
                pos = cursor + incl - mi
                plsc.store_scatter(msrc, [pos], sv, mask=m)
                plsc.store_scatter(mloc, [pos], d - lo, mask=m)
                return cursor + incl[15]

            def drain_full(b, _):
                base = b * GB
                gather_batch(base)

                def step(j, _):
                    accum_edge(base, j)
                    return 0

                lax.fori_loop(0, GB, step, 0, unroll=2)
                return 0

            def chunk_step(k, cursor):
                pltpu.sync_copy(edge_hbm.at[:, pl.ds(k * CEF, CEF)], ebuf0)
                cursor = lax.fori_loop(0, CEF // 16, scan_group_mk, cursor)
                nb = cursor // GB
                lax.fori_loop(0, nb, drain_full, 0)
                for t in range(GB // 16):
                    sl_to = pl.ds(t * 16, 16)
                    sl_from = pl.ds(nb * GB + t * 16, 16)
                    msrc[sl_to] = msrc[sl_from]
                    mloc[sl_to] = mloc[sl_from]
                return cursor - nb * GB

            cursor = lax.fori_loop(0, E // CEF, chunk_step, jnp.int32(0))
            for t in range(GB // 16):
                msrc[pl.ds(cursor + t * 16, 16)] = jnp.zeros((16,), jnp.int32)
            gather_batch(0)

            def final_step(j, _):
                accum_edge(0, j)
                return 0

            lax.fori_loop(0, cursor, final_step, 0, unroll=False)

        _write_stats(lo, s_hbm, q_hbm, mx_hbm, mn_hbm, deg_hbm,
                     st_s, st_q, st_mx, st_mn, degv, sem_l,
                     with_deg=with_deg)
        return 0

    lax.fori_loop(0, PASSES, run_pass, 0)


def _replay_body_nodeg(a_hbm, edge_hbm, lsrc_hbm, lloc_hbm, cnt_hbm,
                       s_hbm, q_hbm, mx_hbm, mn_hbm,
                       ebuf0, mloc, msrc, rowbuf, rowbuf1,
                       st_s, st_q, st_mx, st_mn, degv, cntbuf,
                       sem_e0, sem_g, sem_g1, sem_l):
    _replay_body(a_hbm, edge_hbm, lsrc_hbm, lloc_hbm, cnt_hbm,
                 s_hbm, q_hbm, mx_hbm, mn_hbm, None,
                 ebuf0, mloc, msrc, rowbuf, rowbuf1,
                 st_s, st_q, st_mx, st_mn, degv, cntbuf,
                 sem_e0, sem_g, sem_g1, sem_l, with_deg=False)


_STATS_OUT = [
    jax.ShapeDtypeStruct((N_PAD, D), jnp.float32),
    jax.ShapeDtypeStruct((N_PAD, D), jnp.float32),
    jax.ShapeDtypeStruct((N_PAD, D), jnp.float32),
    jax.ShapeDtypeStruct((N_PAD, D), jnp.float32),
    jax.ShapeDtypeStruct((N_PAD * 16,), jnp.float32),
]

_STATS_SCRATCH = [
    pltpu.VMEM((MBUF,), jnp.int32),
    pltpu.VMEM((MBUF,), jnp.int32),
    pltpu.VMEM((GB, D), jnp.float32),
    pltpu.VMEM((NODES_PER, D), jnp.float32),
    pltpu.VMEM((NODES_PER, D), jnp.float32),
    pltpu.VMEM((NODES_PER, D), jnp.float32),
    pltpu.VMEM((NODES_PER, D), jnp.float32),
    pltpu.VMEM((NODES_PER * 16,), jnp.float32),
    pltpu.VMEM((16,), jnp.int32),
]

_stats_replay = pl.kernel(
    _replay_body,
    out_type=_STATS_OUT,
    compiler_params=pltpu.CompilerParams(needs_layout_passes=False),
    mesh=plsc.VectorSubcoreMesh(core_axis_name="c", subcore_axis_name="s"),
    scratch_types=[
        pltpu.VMEM((2, CEF), jnp.int32),
        pltpu.VMEM((MBUF_R,), jnp.int32),
        pltpu.VMEM((MBUF_R,), jnp.int32),
        pltpu.VMEM((GB, D), jnp.float32),
        pltpu.VMEM((GB, D), jnp.float32),
        pltpu.VMEM((NODES_PER, D), jnp.float32),
        pltpu.VMEM((NODES_PER, D), jnp.float32),
        pltpu.VMEM((NODES_PER, D), jnp.float32),
        pltpu.VMEM((NODES_PER, D), jnp.float32),
        pltpu.VMEM((NODES_PER * 16,), jnp.float32),
        pltpu.VMEM((16,), jnp.int32),
        pltpu.SemaphoreType.DMA,
        pltpu.SemaphoreType.DMA,
        pltpu.SemaphoreType.DMA,
        pltpu.SemaphoreType.DMA,
    ],
)


_stats_replay_nodeg = pl.kernel(
    _replay_body_nodeg,
    out_type=_STATS_OUT[:4],
    compiler_params=pltpu.CompilerParams(needs_layout_passes=False),
    mesh=plsc.VectorSubcoreMesh(core_axis_name="c", subcore_axis_name="s"),
    scratch_types=[
        pltpu.VMEM((2, CEF), jnp.int32),
        pltpu.VMEM((MBUF_R,), jnp.int32),
        pltpu.VMEM((MBUF_R,), jnp.int32),
        pltpu.VMEM((GB, D), jnp.float32),
        pltpu.VMEM((GB, D), jnp.float32),
        pltpu.VMEM((NODES_PER, D), jnp.float32),
        pltpu.VMEM((NODES_PER, D), jnp.float32),
        pltpu.VMEM((NODES_PER, D), jnp.float32),
        pltpu.VMEM((NODES_PER, D), jnp.float32),
        pltpu.VMEM((NODES_PER * 16,), jnp.float32),
        pltpu.VMEM((16,), jnp.int32),
        pltpu.SemaphoreType.DMA,
        pltpu.SemaphoreType.DMA,
        pltpu.SemaphoreType.DMA,
        pltpu.SemaphoreType.DMA,
    ],
)


def _stats_from_lists(A, edge_index, lists):
    lsrc, lloc, cnt = lists
    S, Q, Mx, Mn, deg = _stats_replay(A, edge_index, lsrc, lloc, cnt)
    return S[:N], Q[:N], Mx[:N], Mn[:N], deg.reshape(N_PAD, 16)[:N, 0]


def _stats_from_lists_nodeg(A, edge_index, lists):
    lsrc, lloc, cnt = lists
    S, Q, Mx, Mn = _stats_replay_nodeg(A, edge_index, lsrc, lloc, cnt)
    return S[:N], Q[:N], Mx[:N], Mn[:N]


def _stats_blocks(h_ref, b_ref, s_ref, q_ref, mx_ref, mn_ref, deg_ref):
    deg = deg_ref[...]
    B = b_ref[...]
    S = s_ref[...]
    denom = jnp.maximum(deg, 1.0)
    mean = (S + deg * B) / denom
    sq = (q_ref[...] + 2.0 * B * S + deg * B * B) / denom
    std = jnp.sqrt(jnp.maximum(sq - mean * mean, 0.0) + 1e-5)
    pos = deg > 0.0
    mx = jnp.where(pos, mx_ref[...] + B, 0.0)
    mn = jnp.where(pos, mn_ref[...] + B, 0.0)
    logd = jnp.log(deg + 1.0)
    amp = logd / DELTA_CONST
    att = jnp.where(pos, DELTA_CONST / jnp.maximum(logd, 1e-12), 1.0)
    h = h_ref[...]
    return h, mean, mx, mn, std, amp, att


def _pna_update(h, mean, mx, mn, std, amp, att, U, Ub, mixW, mixb):
    def dot(x, w):
        return jnp.dot(x, w, preferred_element_type=jnp.float32)

    acc = dot(h, U[0:D])
    accI = dot(mean, U[D:2 * D]) + dot(mx, U[2 * D:3 * D]) \
        + dot(mn, U[3 * D:4 * D]) + dot(std, U[4 * D:5 * D])
    accA = dot(mean, U[5 * D:6 * D]) + dot(mx, U[6 * D:7 * D]) \
        + dot(mn, U[7 * D:8 * D]) + dot(std, U[8 * D:9 * D])
    accT = dot(mean, U[9 * D:10 * D]) + dot(mx, U[10 * D:11 * D]) \
        + dot(mn, U[11 * D:12 * D]) + dot(std, U[12 * D:13 * D])
    pre = acc + accI + amp * accA + att * accT + Ub
    out = dot(pre, mixW) + mixb
    out = jnp.where(out >= 0.0, out, 0.01 * out) + h
    return out


def _combine1_body(h_ref, b_ref, s_ref, q_ref, mx_ref, mn_ref, deg_ref,
                   u_ref, ub_ref, mixw_ref, mixb_ref, out_ref):
    h, mean, mx, mn, std, amp, att = _stats_blocks(
        h_ref, b_ref, s_ref, q_ref, mx_ref, mn_ref, deg_ref)
    out = _pna_update(h, mean, mx, mn, std, amp, att,
                      u_ref[...], ub_ref[...], mixw_ref[...], mixb_ref[...])
    out_ref[...] = jnp.maximum(out, 0.0)


def _combine2_body(h_ref, b_ref, s_ref, q_ref, mx_ref, mn_ref, deg_ref,
                   u_ref, ub_ref, mixw_ref, mixb_ref, fcw_ref, fcb_ref,
                   out_ref, acc_ref):
    i = pl.program_id(0)
    h, mean, mx, mn, std, amp, att = _stats_blocks(
        h_ref, b_ref, s_ref, q_ref, mx_ref, mn_ref, deg_ref)
    out = _pna_update(h, mean, mx, mn, std, amp, att,
                      u_ref[...], ub_ref[...], mixw_ref[...], mixb_ref[...])
    partial = jnp.sum(out, axis=0, keepdims=True)

    @pl.when(i == 0)
    def _():
        acc_ref[...] = partial

    @pl.when(i > 0)
    def _():
        acc_ref[...] = acc_ref[...] + partial

    @pl.when(i == (N // BN) - 1)
    def _():
        hg = acc_ref[...] * (1.0 / N)
        out_ref[...] = jnp.dot(hg, fcw_ref[...],
                               preferred_element_type=jnp.float32) + fcb_ref[...]


def _node_spec():
    return pl.BlockSpec((BN, D), lambda i: (i, 0))


def _fixed(shape):
    return pl.BlockSpec(shape, lambda i: tuple(0 for _ in shape))


def _combine1(h, B, S, Q, Mx, Mn, deg, U_W, U_b, mix_W, mix_b):
    return pl.pallas_call(
        _combine1_body,
        grid=(N // BN,),
        in_specs=[
            _node_spec(), _node_spec(), _node_spec(), _node_spec(),
            _node_spec(), _node_spec(),
            pl.BlockSpec((BN, 1), lambda i: (i, 0)),
            _fixed((13 * D, H)), _fixed((1, H)), _fixed((H, H)), _fixed((1, H)),
        ],
        out_specs=_node_spec(),
        out_shape=jax.ShapeDtypeStruct((N, H), jnp.float32),
    )(h, B, S, Q, Mx, Mn, deg.reshape(N, 1), U_W, U_b.reshape(1, H),
      mix_W, mix_b.reshape(1, H))


def _combine2(h, B, S, Q, Mx, Mn, deg, U_W, U_b, mix_W, mix_b, fc_W, fc_b):
    return pl.pallas_call(
        _combine2_body,
        grid=(N // BN,),
        in_specs=[
            _node_spec(), _node_spec(), _node_spec(), _node_spec(),
            _node_spec(), _node_spec(),
            pl.BlockSpec((BN, 1), lambda i: (i, 0)),
            _fixed((13 * H, H)), _fixed((1, H)), _fixed((H, H)), _fixed((1, H)),
            _fixed((H, C)), _fixed((1, C)),
        ],
        out_specs=_fixed((1, C)),
        out_shape=jax.ShapeDtypeStruct((1, C), jnp.float32),
        scratch_shapes=[pltpu.VMEM((1, H), jnp.float32)],
    )(h, B, S, Q, Mx, Mn, deg.reshape(N, 1), U_W, U_b.reshape(1, H),
      mix_W, mix_b.reshape(1, H), fc_W, fc_b.reshape(1, C))


def kernel(x, edge_index, M1_W, M1_b, U1_W, U1_b, mix1_W, mix1_b,
           M2_W, M2_b, U2_W, U2_b, mix2_W, mix2_b, fc_W, fc_b):
    src = edge_index[0]
    dst = edge_index[1]

    lists = _edge_scan(edge_index)
    A1, B1 = _prep(x, M1_W, M1_b)
    S, Q, Mx, Mn, deg = _stats_from_lists(A1, edge_index, lists)
    h1 = _combine1(x, B1, S, Q, Mx, Mn, deg, U1_W, U1_b, mix1_W, mix1_b)

    A2, B2 = _prep(h1, M2_W, M2_b)
    S, Q, Mx, Mn = _stats_from_lists_nodeg(A2, edge_index, lists)
    return _combine2(h1, B2, S, Q, Mx, Mn, deg, U2_W, U2_b, mix2_W, mix2_b,
                     fc_W, fc_b)
